# Initial kernel scaffold; baseline (speedup 1.0000x reference)
import jax
import jax.numpy as jnp
from jax.experimental import pallas as pl


def kernel(input, vectors, bias):
    raise NotImplementedError("write your pallas kernel here")


# speedup vs baseline: 12.3163x; 12.3163x over previous
import functools

import jax
import jax.numpy as jnp
from jax import lax
from jax.experimental import pallas as pl
from jax.experimental.pallas import tpu as pltpu
from jax.experimental.pallas import tpu_sc as plsc

_MAPPING = (0, 1, 1, 1, 1, 2, 2, 2, 2, 3, 3, 3, 3, 3, 4, 4, 4, 4, 5, 6, 7, 7, 7)
_DISTANCES = (0., 0., 1., 2., 3., 0., 1., 2., 3., 0., 1., 2., 3., 4., 0., 1., 2., 3., 0., 0., 0., 1., 2.)
_NCLS = 23
_NMAJ = 8
_E = 32

_NC = 2
_NS = 16
_NW = _NC * _NS
_ROW = 128


@functools.partial(jax.jit, static_argnums=(3, 4))
def _sc_lookup(idx2d, vectors, bias, per_w, g_rows):
    rows_total = idx2d.shape[0]
    ngroups = per_w // g_rows
    mesh = plsc.VectorSubcoreMesh(core_axis_name="c", subcore_axis_name="s")

    @functools.partial(
        pl.kernel,
        mesh=mesh,
        compiler_params=pltpu.CompilerParams(use_tc_tiling_on_sc=False),
        out_type=[
            jax.ShapeDtypeStruct((rows_total, _ROW, _E), jnp.float32),
            jax.ShapeDtypeStruct((_NCLS, _E), jnp.float32),
        ],
        scratch_types=[
            pltpu.VMEM((_NMAJ, _E), jnp.float32),
            pltpu.VMEM((_NMAJ, _E), jnp.float32),
            pltpu.VMEM((_NCLS, _E), jnp.float32),
            pltpu.VMEM((g_rows, _ROW), jnp.int32),
            pltpu.VMEM((g_rows, _ROW, _E), jnp.float32),
            pltpu.SemaphoreType.DMA,
        ],
    )
    def k(idx_hbm, vec_hbm, bias_hbm, out_hbm, table_hbm,
          vec_v, bias_v, table_v, idx_v, rows_v, sem):
        wid = lax.axis_index("s") * _NC + lax.axis_index("c")
        pltpu.sync_copy(vec_hbm, vec_v)
        pltpu.sync_copy(bias_hbm, bias_v)
        for c in range(_NCLS):
            m = _MAPPING[c]
            d = _DISTANCES[c]
            for h in range(_E // 16):
                sl = pl.ds(h * 16, 16)
                table_v[c, sl] = bias_v[m, sl] + d * vec_v[m, sl]
        pltpu.sync_copy(table_v, table_hbm)
        base = wid * per_w
        for g in range(ngroups):
            r0 = base + g * g_rows
            pltpu.sync_copy(idx_hbm.at[pl.ds(r0, g_rows)], idx_v)
            copies = [
                pltpu.async_copy(table_hbm.at[idx_v.at[j]], rows_v.at[j], sem)
                for j in range(g_rows)
            ]
            for cp in copies:
                cp.wait()
            pltpu.sync_copy(rows_v, out_hbm.at[pl.ds(r0, g_rows)])

    out, _ = k(idx2d, vectors, bias)
    return out


def kernel(input, vectors, bias):
    shape = input.shape
    n = input.size
    assert n % (_NW * _ROW) == 0
    idx2d = input.reshape(-1, _ROW).astype(jnp.int32)
    per_w = idx2d.shape[0] // _NW
    g_rows = 24 if per_w % 24 == 0 else 8
    out = _sc_lookup(idx2d, vectors, bias, per_w, g_rows)
    return out.reshape(*shape, _E)


# speedup vs baseline: 17.2979x; 1.4045x over previous
import functools

import jax
import jax.numpy as jnp
from jax import lax
from jax.experimental import pallas as pl
from jax.experimental.pallas import tpu as pltpu
from jax.experimental.pallas import tpu_sc as plsc

_MAPPING = (0, 1, 1, 1, 1, 2, 2, 2, 2, 3, 3, 3, 3, 3, 4, 4, 4, 4, 5, 6, 7, 7, 7)
_DISTANCES = (0., 0., 1., 2., 3., 0., 1., 2., 3., 0., 1., 2., 3., 4., 0., 1., 2., 3., 0., 0., 0., 1., 2.)
_NCLS = 23
_NMAJ = 8
_E = 32

_NC = 2
_NS = 16
_NW = _NC * _NS
_CH = 1728


@functools.partial(jax.jit, static_argnums=(3,))
def _sc_lookup(idx_flat, vectors, bias, per_w):
    n = idx_flat.shape[0]
    nchunk = per_w // _CH
    mesh = plsc.VectorSubcoreMesh(core_axis_name="c", subcore_axis_name="s")

    @functools.partial(
        pl.kernel,
        mesh=mesh,
        compiler_params=pltpu.CompilerParams(needs_layout_passes=False),
        out_type=jax.ShapeDtypeStruct((n * _E,), jnp.float32),
        scratch_types=[
            pltpu.VMEM((_NMAJ, _E), jnp.float32),
            pltpu.VMEM((_NMAJ, _E), jnp.float32),
            pltpu.VMEM((_NCLS * _E,), jnp.float32),
            pltpu.VMEM((_CH,), jnp.int32),
            pltpu.VMEM((_CH * _E,), jnp.float32),
            pltpu.VMEM((_CH * _E,), jnp.float32),
            pltpu.SemaphoreType.DMA,
            pltpu.SemaphoreType.DMA,
        ],
    )
    def k(idx_hbm, vec_hbm, bias_hbm, out_hbm,
          vec_v, bias_v, table_v, idx_v, ob0, ob1, sem0, sem1):
        wid = lax.axis_index("s") * _NC + lax.axis_index("c")
        pltpu.sync_copy(vec_hbm, vec_v)
        pltpu.sync_copy(bias_hbm, bias_v)
        for c in range(_NCLS):
            m = _MAPPING[c]
            d = _DISTANCES[c]
            for h in range(_E // 16):
                table_v[pl.ds(c * _E + h * 16, 16)] = (
                    bias_v[m, pl.ds(h * 16, 16)] + d * vec_v[m, pl.ds(h * 16, 16)]
                )
        base = wid * per_w
        lane = lax.iota(jnp.int32, 16)
        row_stride = lane * _E
        obufs = (ob0, ob1)
        pending = [None, None]

        def compute(ob):
            def body(g, _):
                c16 = idx_v[pl.ds(g * 16, 16)]
                gbase = c16 * _E
                sbase = row_stride + g * (16 * _E)
                for j in range(_E):
                    v = plsc.load_gather(table_v, [gbase + j])
                    plsc.store_scatter(ob, [sbase + j], v)
                return 0

            lax.fori_loop(0, _CH // 16, body, 0, unroll=False)

        for kk in range(nchunk):
            b = kk % 2
            ob, sem = obufs[b], (sem0, sem1)[b]
            pltpu.sync_copy(idx_hbm.at[pl.ds(base + kk * _CH, _CH)], idx_v)
            if pending[b] is not None:
                pending[b].wait()
            compute(ob)
            pending[b] = pltpu.async_copy(
                ob, out_hbm.at[pl.ds((base + kk * _CH) * _E, _CH * _E)], sem)
        for p in pending:
            if p is not None:
                p.wait()

    return k(idx_flat, vectors, bias)


def kernel(input, vectors, bias):
    shape = input.shape
    n = input.size
    assert n % (_NW * _CH) == 0
    idx_flat = input.reshape(-1).astype(jnp.int32)
    per_w = n // _NW
    out = _sc_lookup(idx_flat, vectors, bias, per_w)
    return out.reshape(*shape, _E)


# speedup vs baseline: 105.1550x; 6.0791x over previous
import functools

import jax
import jax.numpy as jnp
from jax import lax
from jax.experimental import pallas as pl
from jax.experimental.pallas import tpu as pltpu
from jax.experimental.pallas import tpu_sc as plsc

_MAPPING = (0, 1, 1, 1, 1, 2, 2, 2, 2, 3, 3, 3, 3, 3, 4, 4, 4, 4, 5, 6, 7, 7, 7)
_DISTANCES = (0., 0., 1., 2., 3., 0., 1., 2., 3., 0., 1., 2., 3., 4., 0., 1., 2., 3., 0., 0., 0., 1., 2.)
_NCLS = 23
_NMAJ = 8
_E = 32

_NC = 2
_NS = 16
_NW = _NC * _NS
_TSTRIDE = 737
_CB = 8


@functools.partial(jax.jit, static_argnums=(3, 4))
def _sc_lookup(idx2d, vectors, bias, per_w, w):
    rows_total = idx2d.shape[0]
    nchunk = per_w // _CB
    mesh = plsc.VectorSubcoreMesh(core_axis_name="c", subcore_axis_name="s")

    @functools.partial(
        pl.kernel,
        mesh=mesh,
        compiler_params=pltpu.CompilerParams(needs_layout_passes=False),
        out_type=jax.ShapeDtypeStruct((rows_total, _E, w), jnp.float32),
        scratch_types=[
            pltpu.VMEM((_NMAJ, _E), jnp.float32),
            pltpu.VMEM((_NMAJ, _E), jnp.float32),
            pltpu.VMEM((16 * _TSTRIDE,), jnp.float32),
            pltpu.VMEM((_CB, w), jnp.int32),
            pltpu.VMEM((_CB, _E, w), jnp.float32),
            pltpu.VMEM((_CB, _E, w), jnp.float32),
            pltpu.SemaphoreType.DMA,
            pltpu.SemaphoreType.DMA,
        ],
    )
    def k(idx_hbm, vec_hbm, bias_hbm, out_hbm,
          vec_v, bias_v, table_v, idx_v, ob0, ob1, sem0, sem1):
        wid = lax.axis_index("s") * _NC + lax.axis_index("c")
        lane = lax.iota(jnp.int32, 16)
        pltpu.sync_copy(vec_hbm, vec_v)
        pltpu.sync_copy(bias_hbm, bias_v)
        for c in range(_NCLS):
            m = _MAPPING[c]
            d = _DISTANCES[c]
            for h in range(_E // 16):
                val = bias_v[m, pl.ds(h * 16, 16)] + d * vec_v[m, pl.ds(h * 16, 16)]
                for rep in range(16):
                    plsc.store_scatter(
                        table_v, [lane + (rep * _TSTRIDE + c * _E + h * 16)], val)
        lane_rep = lane * _TSTRIDE
        base_row = wid * per_w
        obufs = (ob0, ob1)
        sems = (sem0, sem1)

        def expand(ob):
            def row_body(r, carry):
                for g in range(w // 16):
                    c16 = idx_v[r, pl.ds(g * 16, 16)]
                    bvec = lane_rep + c16 * _E
                    for e in range(_E):
                        v = plsc.load_gather(table_v, [bvec + e])
                        ob[r, e, pl.ds(g * 16, 16)] = v
                return carry

            lax.fori_loop(0, _CB, row_body, 0, unroll=False)

        def outer(kk, carry):
            for b in range(2):
                ob, sem = obufs[b], sems[b]
                row0 = base_row + (kk * 2 + b) * _CB
                pltpu.sync_copy(idx_hbm.at[pl.ds(row0, _CB)], idx_v)

                @pl.when(kk > 0)
                def _drain():
                    pltpu.make_async_copy(
                        out_hbm.at[pl.ds(base_row, _CB)], ob, sem).wait()

                expand(ob)
                pltpu.async_copy(ob, out_hbm.at[pl.ds(row0, _CB)], sem)
            return carry

        lax.fori_loop(0, nchunk // 2, outer, 0, unroll=False)
        for b in range(2):
            pltpu.make_async_copy(
                out_hbm.at[pl.ds(base_row, _CB)], obufs[b], sems[b]).wait()

    return k(idx2d, vectors, bias)


def kernel(input, vectors, bias):
    shape = input.shape
    w = shape[-1]
    rows_total = input.size // w
    assert w % 16 == 0 and rows_total % (_NW * _CB) == 0
    idx2d = input.reshape(rows_total, w).astype(jnp.int32)
    per_w = rows_total // _NW
    out = _sc_lookup(idx2d, vectors, bias, per_w, w)
    out = out.reshape(*shape[:-1], _E, w)
    perm = list(range(len(shape) - 1)) + [len(shape), len(shape) - 1]
    return out.transpose(*perm)


# speedup vs baseline: 332.8206x; 3.1650x over previous
import functools

import jax
import jax.numpy as jnp
from jax import lax
from jax.experimental import pallas as pl
from jax.experimental.pallas import tpu as pltpu
from jax.experimental.pallas import tpu_sc as plsc

_MAPPING = (0, 1, 1, 1, 1, 2, 2, 2, 2, 3, 3, 3, 3, 3, 4, 4, 4, 4, 5, 6, 7, 7, 7)
_DISTANCES = (0., 0., 1., 2., 3., 0., 1., 2., 3., 0., 1., 2., 3., 4., 0., 1., 2., 3., 0., 0., 0., 1., 2.)
_NCLS = 23
_NMAJ = 8
_E = 32

_NC = 2
_NS = 16
_NW = _NC * _NS
_TSTRIDE = 737
_CB = 8


@functools.partial(jax.jit, static_argnums=(3, 4))
def _sc_lookup(idx2d, vectors, bias, per_w, w):
    rows_total = idx2d.shape[0]
    nchunk = per_w // _CB
    mesh = plsc.VectorSubcoreMesh(core_axis_name="c", subcore_axis_name="s")

    @functools.partial(
        pl.kernel,
        mesh=mesh,
        compiler_params=pltpu.CompilerParams(needs_layout_passes=False),
        out_type=jax.ShapeDtypeStruct((rows_total, _E, w), jnp.float32),
        scratch_types=[
            pltpu.VMEM((_NMAJ, _E), jnp.float32),
            pltpu.VMEM((_NMAJ, _E), jnp.float32),
            pltpu.VMEM((16 * _TSTRIDE,), jnp.float32),
            pltpu.VMEM((_CB, w), jnp.int32),
            pltpu.VMEM((_CB, w), jnp.int32),
            pltpu.VMEM((_CB, _E, w), jnp.float32),
            pltpu.VMEM((_CB, _E, w), jnp.float32),
            pltpu.SemaphoreType.DMA,
            pltpu.SemaphoreType.DMA,
            pltpu.SemaphoreType.DMA,
            pltpu.SemaphoreType.DMA,
        ],
    )
    def k(idx_hbm, vec_hbm, bias_hbm, out_hbm,
          vec_v, bias_v, table_v, idx0, idx1, ob0, ob1,
          sem0, sem1, isem0, isem1):
        wid = lax.axis_index("s") * _NC + lax.axis_index("c")
        lane = lax.iota(jnp.int32, 16)
        pltpu.sync_copy(vec_hbm, vec_v)
        pltpu.sync_copy(bias_hbm, bias_v)
        for c in range(_NCLS):
            m = _MAPPING[c]
            d = _DISTANCES[c]
            for h in range(_E // 16):
                val = bias_v[m, pl.ds(h * 16, 16)] + d * vec_v[m, pl.ds(h * 16, 16)]
                for rep in range(16):
                    plsc.store_scatter(
                        table_v, [lane + (rep * _TSTRIDE + c * _E + h * 16)], val)
        lane_rep = lane * _TSTRIDE
        base_row = wid * per_w
        obufs = (ob0, ob1)
        sems = (sem0, sem1)
        ibufs = (idx0, idx1)
        isems = (isem0, isem1)
        ngrp = w // 16
        _D = 6

        def expand(idx_v, ob):
            def row_body(r, carry):
                bases = []
                for g in range(ngrp):
                    c16 = idx_v[r, pl.ds(g * 16, 16)]
                    bases.append(lane_rep + c16 * _E)
                pend = []
                for g in range(ngrp):
                    for e in range(_E):
                        v = plsc.load_gather(table_v, [bases[g] + e])
                        pend.append((g, e, v))
                        if len(pend) > _D:
                            g2, e2, v2 = pend.pop(0)
                            ob[r, e2, pl.ds(g2 * 16, 16)] = v2
                for g2, e2, v2 in pend:
                    ob[r, e2, pl.ds(g2 * 16, 16)] = v2
                return carry

            lax.fori_loop(0, _CB, row_body, 0, unroll=False)

        pltpu.async_copy(idx_hbm.at[pl.ds(base_row, _CB)], idx0, isem0)

        def outer(kk, carry):
            for b in range(2):
                ob, sem = obufs[b], sems[b]
                ch = kk * 2 + b
                row0 = base_row + ch * _CB

                @pl.when(ch + 1 < nchunk)
                def _prefetch():
                    pltpu.async_copy(
                        idx_hbm.at[pl.ds(row0 + _CB, _CB)], ibufs[1 - b],
                        isems[1 - b])

                pltpu.make_async_copy(
                    idx_hbm.at[pl.ds(base_row, _CB)], ibufs[b], isems[b]).wait()

                @pl.when(kk > 0)
                def _drain():
                    pltpu.make_async_copy(
                        out_hbm.at[pl.ds(base_row, _CB)], ob, sem).wait()

                expand(ibufs[b], ob)
                pltpu.async_copy(ob, out_hbm.at[pl.ds(row0, _CB)], sem)
            return carry

        lax.fori_loop(0, nchunk // 2, outer, 0, unroll=False)
        for b in range(2):
            pltpu.make_async_copy(
                out_hbm.at[pl.ds(base_row, _CB)], obufs[b], sems[b]).wait()

    return k(idx2d, vectors, bias)


def kernel(input, vectors, bias):
    shape = input.shape
    w = shape[-1]
    rows_total = input.size // w
    assert w % 16 == 0 and rows_total % (_NW * _CB) == 0
    idx2d = input.reshape(rows_total, w).astype(jnp.int32)
    per_w = rows_total // _NW
    out = _sc_lookup(idx2d, vectors, bias, per_w, w)
    out = out.reshape(*shape[:-1], _E, w)
    perm = list(range(len(shape) - 1)) + [len(shape), len(shape) - 1]
    return out.transpose(*perm)
